# Initial kernel scaffold; baseline (speedup 1.0000x reference)
import jax
import jax.numpy as jnp
from jax.experimental import pallas as pl


def kernel(ctx_embed, pred_len, query_experts, query_pos, pred_len_emb, latents, W_lat_q, W_ctx_k, W_ctx_v, W_lat_out, W_step_q, W_lat_k, W_lat_v, W_step_out, W_g1, b_g1, W_g2, b_g2):
    raise NotImplementedError("write your pallas kernel here")


# speedup vs baseline: 1.6240x; 1.6240x over previous
import jax
import jax.numpy as jnp
from jax.experimental import pallas as pl

B, T, D = 4, 2048, 1024
E, L, P, TOPK = 16, 8, 512, 2

_SQRT_HALF = 0.7071067811865476


def _dot(a, b):
    return jax.lax.dot_general(a, b, (((1,), (0,)), ((), ())))


def _dot_t(a, b):
    return jax.lax.dot_general(a, b, (((1,), (1,)), ((), ())))


def _softmax_rows(s):
    m = jnp.max(s, axis=-1, keepdims=True)
    e = jnp.exp(s - m)
    return e / jnp.sum(e, axis=-1, keepdims=True)


def _latctx_body(ctx_ref, lat_ref, wlq_ref, wck_ref, wcv_ref, u_ref):
    ctx = ctx_ref[0]
    q1 = _dot_t(lat_ref[...], wlq_ref[...])
    k1 = _dot_t(ctx, wck_ref[...])
    s = _dot_t(q1, k1) / 32.0
    p = _softmax_rows(s)
    v1 = _dot_t(ctx, wcv_ref[...])
    u_ref[0] = _dot(p, v1)


def _gate_body(u_ref, qpos_ref, lenv_ref, wlo_ref, wlk_ref, wlv_ref,
               wsq_ref, wso_ref, wg1_ref, bg1_ref, wg2_ref, bg2_ref, w_ref):
    lat_ctx = _dot_t(u_ref[0], wlo_ref[...])
    k2 = _dot_t(lat_ctx, wlk_ref[...])
    v2 = _dot_t(lat_ctx, wlv_ref[...])
    sb = qpos_ref[...] + lenv_ref[...]
    q2 = _dot_t(sb, wsq_ref[...])
    s2 = _dot_t(q2, k2) / 32.0
    p2 = _softmax_rows(s2)
    a2 = _dot(p2, v2)
    sc = _dot_t(a2, wso_ref[...])
    gi = jnp.concatenate([sb, sc], axis=-1)
    z = _dot_t(gi, wg1_ref[...]) + bg1_ref[...]
    h = 0.5 * z * (1.0 + jax.lax.erf(z * _SQRT_HALF))
    logits = _dot_t(h, wg2_ref[...]) + bg2_ref[...]
    lane = jax.lax.broadcasted_iota(jnp.int32, (P, E), 1)
    m1 = jnp.max(logits, axis=-1, keepdims=True)
    i1 = jnp.min(jnp.where(logits == m1, lane, E), axis=-1, keepdims=True)
    masked = jnp.where(lane == i1, -jnp.inf, logits)
    m2v = jnp.max(masked, axis=-1, keepdims=True)
    i2 = jnp.min(jnp.where(masked == m2v, lane, E), axis=-1, keepdims=True)
    eb = jnp.exp(m2v - m1)
    denom = 1.0 + eb
    w = jnp.where(lane == i1, 1.0 / denom,
                  jnp.where(lane == i2, eb / denom, 0.0))
    w_ref[0] = w


def _combine_body(w_ref, qe_ref, out_ref):
    acc = w_ref[:, :, 0:1] * qe_ref[0][None]
    for e in range(1, E):
        acc = acc + w_ref[:, :, e:e + 1] * qe_ref[e][None]
    out_ref[...] = acc


def kernel(ctx_embed, pred_len, query_experts, query_pos, pred_len_emb,
           latents, W_lat_q, W_ctx_k, W_ctx_v, W_lat_out, W_step_q, W_lat_k,
           W_lat_v, W_step_out, W_g1, b_g1, W_g2, b_g2):
    f32 = jnp.float32
    len_vec = jax.lax.dynamic_index_in_dim(pred_len_emb, pred_len, axis=0,
                                           keepdims=True)
    qpos = query_pos[:P]

    wfull = lambda s: pl.BlockSpec(s, lambda b: (0,) * len(s))

    u = pl.pallas_call(
        _latctx_body,
        grid=(B,),
        in_specs=[
            pl.BlockSpec((1, T, D), lambda b: (b, 0, 0)),
            wfull((L, D)), wfull((D, D)), wfull((D, D)), wfull((D, D)),
        ],
        out_specs=pl.BlockSpec((1, L, D), lambda b: (b, 0, 0)),
        out_shape=jax.ShapeDtypeStruct((B, L, D), f32),
    )(ctx_embed, latents, W_lat_q, W_ctx_k, W_ctx_v)

    weights = pl.pallas_call(
        _gate_body,
        grid=(B,),
        in_specs=[
            pl.BlockSpec((1, L, D), lambda b: (b, 0, 0)),
            wfull((P, D)), wfull((1, D)),
            wfull((D, D)), wfull((D, D)), wfull((D, D)),
            wfull((D, D)), wfull((D, D)), wfull((D, 2 * D)),
            wfull((1, D)), wfull((E, D)), wfull((1, E)),
        ],
        out_specs=pl.BlockSpec((1, P, E), lambda b: (b, 0, 0)),
        out_shape=jax.ShapeDtypeStruct((B, P, E), f32),
    )(u, qpos, len_vec, W_lat_out, W_lat_k, W_lat_v, W_step_q, W_step_out,
      W_g1, b_g1.reshape(1, D), W_g2, b_g2.reshape(1, E))

    PC = 128
    queries = pl.pallas_call(
        _combine_body,
        grid=(P // PC,),
        in_specs=[
            pl.BlockSpec((B, PC, E), lambda j: (0, j, 0)),
            pl.BlockSpec((E, PC, D), lambda j: (0, j, 0)),
        ],
        out_specs=pl.BlockSpec((B, PC, D), lambda j: (0, j, 0)),
        out_shape=jax.ShapeDtypeStruct((B, P, D), f32),
    )(weights, query_experts[:, :P, :])

    return queries
